# baseline (speedup 1.0000x reference)
import functools

import jax
import jax.numpy as jnp
from jax.experimental import pallas as pl
from jax.experimental.pallas import tpu as pltpu

ALPHA = 0.2
NEG_BIG = -9e15


def _leaky_relu(v):
    return jnp.where(v >= 0, v, ALPHA * v)


def _elu(v):
    return jnp.where(v > 0, v, jnp.exp(v) - 1.0)


def _gat_block_kernel(x_ref, adj_ref, g_ref, ev1_ref, ev2t_ref, out_ref, *,
                      heads, hidden, pairs):
    n = adj_ref.shape[1]
    ones_col = jnp.ones((n, 1), dtype=jnp.bfloat16)
    for j in range(pairs):
        xv = x_ref[j, 0, :]
        xvb = xv.astype(jnp.bfloat16)
        adj = adj_ref[j, :, :]
        maskb = jnp.where(adj > 0, 1.0, 0.0).astype(jnp.bfloat16)
        e1c = xv[:, None] * ev1_ref[:, :]
        e2r = ev2t_ref[:, :] * xv[None, :]
        rcol = jnp.exp((ALPHA - 1.0) * e1c).astype(jnp.bfloat16)
        exp_e2 = jnp.exp(e2r).astype(jnp.bfloat16)
        exp_a2 = jnp.exp(ALPHA * e2r).astype(jnp.bfloat16)
        gw = xvb[:, None] * g_ref[:, :]
        code = None
        for h in range(heads):
            ua = rcol[:, h:h + 1] * exp_a2[h:h + 1, :]
            p = jnp.maximum(exp_e2[h:h + 1, :], ua) * maskb
            wh1 = jnp.concatenate(
                [gw[:, h * hidden:(h + 1) * hidden], ones_col], axis=1)
            hs = jnp.dot(p, wh1, preferred_element_type=jnp.float32)
            inv = (1.0 / hs[:, hidden:hidden + 1]).astype(jnp.bfloat16)
            hpb = hs[:, :hidden].astype(jnp.bfloat16) * inv
            hp = jnp.where(hpb > 0, hpb, jnp.exp(hpb) - 1.0)
            gated = xvb[:, None] * hp
            cpart = jnp.max(gated, axis=0)
            code = cpart if code is None else jnp.concatenate([code, cpart])
        out_ref[j, 0, :] = code.astype(jnp.float32)


def _tail_kernel(c0_ref, c1_ref, cls_ref, ca1_ref, ca2_ref, dw_ref, db_ref,
                 out_ref):
    cls = cls_ref[0, :]
    q = jnp.sum(cls * ca1_ref[0, :])
    e0 = _leaky_relu(q + jnp.sum(cls * ca2_ref[0, :]))
    e1 = _leaky_relu(q + jnp.sum(c0_ref[:, :] * ca2_ref[0, :][None, :], axis=1))
    e2 = _leaky_relu(q + jnp.sum(c1_ref[:, :] * ca2_ref[0, :][None, :], axis=1))
    m = jnp.maximum(jnp.maximum(e0, e1), e2)
    p0 = jnp.exp(e0 - m)
    p1 = jnp.exp(e1 - m)
    p2 = jnp.exp(e2 - m)
    s = p0 + p1 + p2
    pooled = (p0[:, None] * cls[None, :] + p1[:, None] * c0_ref[:, :]
              + p2[:, None] * c1_ref[:, :]) / s[:, None]
    out = jnp.dot(pooled, dw_ref[:, :], preferred_element_type=jnp.float32)
    out_ref[:, :] = _elu(out + db_ref[0, :][None, :])


def kernel(x, adjs, embedding_weight, W0, a0, cls_weight, cls_a, dense_W,
           dense_b):
    k, b, n = x.shape
    heads, embed, hidden = W0.shape
    d = heads * hidden
    pen = dense_W.shape[1]

    x_r = x.reshape(k * b, 1, n)
    adj_r = adjs.reshape(k * b, n, n)
    w_cat = jnp.transpose(W0, (1, 0, 2)).reshape(embed, d)
    a1s = a0[:, :hidden, 0]
    a2s = a0[:, hidden:, 0]
    v1 = jnp.einsum('hef,hf->eh', W0, a1s)
    v2 = jnp.einsum('hef,hf->he', W0, a2s)
    g_bf = (embedding_weight @ w_cat).astype(jnp.bfloat16)
    ev1 = embedding_weight @ v1
    ev2t = (embedding_weight @ v2.T).T

    pairs = 16
    codes = pl.pallas_call(
        functools.partial(_gat_block_kernel, heads=heads, hidden=hidden,
                          pairs=pairs),
        grid=(k * b // pairs,),
        in_specs=[
            pl.BlockSpec((pairs, 1, n), lambda i: (i, 0, 0)),
            pl.BlockSpec((pairs, n, n), lambda i: (i, 0, 0)),
            pl.BlockSpec((n, d), lambda i: (0, 0)),
            pl.BlockSpec((n, heads), lambda i: (0, 0)),
            pl.BlockSpec((heads, n), lambda i: (0, 0)),
        ],
        out_specs=pl.BlockSpec((pairs, 1, d), lambda i: (i, 0, 0)),
        out_shape=jax.ShapeDtypeStruct((k * b, 1, d), jnp.float32),
        compiler_params=pltpu.CompilerParams(
            dimension_semantics=("parallel",)),
    )(x_r, adj_r, g_bf, ev1, ev2t)

    codes = codes.reshape(k, b, d)
    out = pl.pallas_call(
        _tail_kernel,
        out_shape=jax.ShapeDtypeStruct((b, pen), jnp.float32),
    )(codes[0], codes[1], cls_weight.reshape(1, d),
      cls_a[:d, 0].reshape(1, d), cls_a[d:, 0].reshape(1, d),
      dense_W, dense_b.reshape(1, pen))
    return out


# speedup vs baseline: 1.0161x; 1.0161x over previous
import functools

import jax
import jax.numpy as jnp
from jax.experimental import pallas as pl
from jax.experimental.pallas import tpu as pltpu

ALPHA = 0.2
NEG_BIG = -9e15


def _leaky_relu(v):
    return jnp.where(v >= 0, v, ALPHA * v)


def _elu(v):
    return jnp.where(v > 0, v, jnp.exp(v) - 1.0)


def _gat_block_kernel(x_ref, adj_ref, g_ref, ev1_ref, ev2t_ref, out_ref, *,
                      heads, hidden, pairs):
    n = adj_ref.shape[1]
    ones_col = jnp.ones((n, 1), dtype=jnp.bfloat16)
    for j in range(pairs):
        xv = x_ref[j, 0, :]
        xvb = xv.astype(jnp.bfloat16)
        adj = adj_ref[j, :, :]
        maskb = jnp.where(adj > 0, 1.0, 0.0).astype(jnp.bfloat16)
        e1c = xv[:, None] * ev1_ref[:, :]
        e2r = ev2t_ref[:, :] * xv[None, :]
        rcol = jnp.exp((ALPHA - 1.0) * e1c).astype(jnp.bfloat16)
        exp_e2 = jnp.exp(e2r).astype(jnp.bfloat16)
        exp_a2 = jnp.exp(ALPHA * e2r).astype(jnp.bfloat16)
        gw = xvb[:, None] * g_ref[:, :]
        code = None
        for h in range(heads):
            ua = rcol[:, h:h + 1] * exp_a2[h:h + 1, :]
            p = jnp.maximum(exp_e2[h:h + 1, :], ua) * maskb
            wh1 = jnp.concatenate(
                [gw[:, h * hidden:(h + 1) * hidden], ones_col], axis=1)
            hs = jnp.dot(p, wh1, preferred_element_type=jnp.float32)
            inv = (1.0 / hs[:, hidden:hidden + 1]).astype(jnp.bfloat16)
            hpb = hs[:, :hidden].astype(jnp.bfloat16) * inv
            hp = jnp.where(hpb > 0, hpb, jnp.exp(hpb) - 1.0)
            gated = xvb[:, None] * hp
            cpart = jnp.max(gated, axis=0)
            code = cpart if code is None else jnp.concatenate([code, cpart])
        out_ref[j, 0, :] = code.astype(jnp.float32)


def _tail_kernel(c0_ref, c1_ref, cls_ref, ca1_ref, ca2_ref, dw_ref, db_ref,
                 out_ref):
    cls = cls_ref[0, :]
    q = jnp.sum(cls * ca1_ref[0, :])
    e0 = _leaky_relu(q + jnp.sum(cls * ca2_ref[0, :]))
    e1 = _leaky_relu(q + jnp.sum(c0_ref[:, :] * ca2_ref[0, :][None, :], axis=1))
    e2 = _leaky_relu(q + jnp.sum(c1_ref[:, :] * ca2_ref[0, :][None, :], axis=1))
    m = jnp.maximum(jnp.maximum(e0, e1), e2)
    p0 = jnp.exp(e0 - m)
    p1 = jnp.exp(e1 - m)
    p2 = jnp.exp(e2 - m)
    s = p0 + p1 + p2
    pooled = (p0[:, None] * cls[None, :] + p1[:, None] * c0_ref[:, :]
              + p2[:, None] * c1_ref[:, :]) / s[:, None]
    out = jnp.dot(pooled, dw_ref[:, :], preferred_element_type=jnp.float32)
    out_ref[:, :] = _elu(out + db_ref[0, :][None, :])


def kernel(x, adjs, embedding_weight, W0, a0, cls_weight, cls_a, dense_W,
           dense_b):
    k, b, n = x.shape
    heads, embed, hidden = W0.shape
    d = heads * hidden
    pen = dense_W.shape[1]

    x_r = x.reshape(k * b, 1, n)
    adj_r = adjs.reshape(k * b, n, n)
    w_cat = jnp.transpose(W0, (1, 0, 2)).reshape(embed, d)
    a1s = a0[:, :hidden, 0]
    a2s = a0[:, hidden:, 0]
    v1 = jnp.einsum('hef,hf->eh', W0, a1s)
    v2 = jnp.einsum('hef,hf->he', W0, a2s)
    g_bf = (embedding_weight @ w_cat).astype(jnp.bfloat16)
    ev1 = embedding_weight @ v1
    ev2t = (embedding_weight @ v2.T).T

    pairs = 8
    codes = pl.pallas_call(
        functools.partial(_gat_block_kernel, heads=heads, hidden=hidden,
                          pairs=pairs),
        grid=(k * b // pairs,),
        in_specs=[
            pl.BlockSpec((pairs, 1, n), lambda i: (i, 0, 0)),
            pl.BlockSpec((pairs, n, n), lambda i: (i, 0, 0)),
            pl.BlockSpec((n, d), lambda i: (0, 0)),
            pl.BlockSpec((n, heads), lambda i: (0, 0)),
            pl.BlockSpec((heads, n), lambda i: (0, 0)),
        ],
        out_specs=pl.BlockSpec((pairs, 1, d), lambda i: (i, 0, 0)),
        out_shape=jax.ShapeDtypeStruct((k * b, 1, d), jnp.float32),
        compiler_params=pltpu.CompilerParams(
            dimension_semantics=("arbitrary",)),
    )(x_r, adj_r, g_bf, ev1, ev2t)

    codes = codes.reshape(k, b, d)
    out = pl.pallas_call(
        _tail_kernel,
        out_shape=jax.ShapeDtypeStruct((b, pen), jnp.float32),
    )(codes[0], codes[1], cls_weight.reshape(1, d),
      cls_a[:d, 0].reshape(1, d), cls_a[d:, 0].reshape(1, d),
      dense_W, dense_b.reshape(1, pen))
    return out


# speedup vs baseline: 1.0593x; 1.0424x over previous
import functools

import jax
import jax.numpy as jnp
from jax.experimental import pallas as pl
from jax.experimental.pallas import tpu as pltpu

ALPHA = 0.2
NEG_BIG = -9e15


def _leaky_relu(v):
    return jnp.where(v >= 0, v, ALPHA * v)


def _elu(v):
    return jnp.where(v > 0, v, jnp.exp(v) - 1.0)


def _gat_block_kernel(x_ref, adj_ref, g_ref, ev1_ref, ev2t_ref, cls_ref,
                      ca1_ref, ca2_ref, dw_ref, db_ref, codes_ref, out_ref,
                      *, heads, hidden, pairs, nb):
    n = adj_ref.shape[1]
    i = pl.program_id(0)
    base = i * pairs
    ones_col = jnp.ones((n, 1), dtype=jnp.bfloat16)
    for j in range(pairs):
        xv = x_ref[j, 0, :]
        xvb = xv.astype(jnp.bfloat16)
        adj = adj_ref[j, :, :]
        maskb = jnp.where(adj > 0, 1.0, 0.0).astype(jnp.bfloat16)
        e1c = xv[:, None] * ev1_ref[:, :]
        e2r = ev2t_ref[:, :] * xv[None, :]
        rcol = jnp.exp((ALPHA - 1.0) * e1c).astype(jnp.bfloat16)
        exp_e2 = jnp.exp(e2r).astype(jnp.bfloat16)
        exp_a2 = jnp.exp(ALPHA * e2r).astype(jnp.bfloat16)
        gw = xvb[:, None] * g_ref[:, :]
        code = None
        for h in range(heads):
            ua = rcol[:, h:h + 1] * exp_a2[h:h + 1, :]
            p = jnp.maximum(exp_e2[h:h + 1, :], ua) * maskb
            wh1 = jnp.concatenate(
                [gw[:, h * hidden:(h + 1) * hidden], ones_col], axis=1)
            hs = jnp.dot(p, wh1, preferred_element_type=jnp.float32)
            inv = (1.0 / hs[:, hidden:hidden + 1]).astype(jnp.bfloat16)
            hpb = hs[:, :hidden].astype(jnp.bfloat16) * inv
            hp = jnp.where(hpb > 0, hpb, jnp.exp(hpb) - 1.0)
            gated = xvb[:, None] * hp
            cpart = jnp.max(gated, axis=0)
            code = cpart if code is None else jnp.concatenate([code, cpart])
        codes_ref[base + j, :] = code.astype(jnp.float32)

    @pl.when(i == pl.num_programs(0) - 1)
    def _tail():
        cls = cls_ref[0, :]
        c0 = codes_ref[0:nb, :]
        c1 = codes_ref[nb:2 * nb, :]
        q = jnp.sum(cls * ca1_ref[0, :])
        e0 = _leaky_relu(q + jnp.sum(cls * ca2_ref[0, :]))
        e1 = _leaky_relu(q + jnp.sum(c0 * ca2_ref[0, :][None, :], axis=1))
        e2 = _leaky_relu(q + jnp.sum(c1 * ca2_ref[0, :][None, :], axis=1))
        m = jnp.maximum(jnp.maximum(e0, e1), e2)
        p0 = jnp.exp(e0 - m)
        p1 = jnp.exp(e1 - m)
        p2 = jnp.exp(e2 - m)
        s = p0 + p1 + p2
        pooled = (p0[:, None] * cls[None, :] + p1[:, None] * c0
                  + p2[:, None] * c1) / s[:, None]
        o = jnp.dot(pooled, dw_ref[:, :], preferred_element_type=jnp.float32)
        out_ref[:, :] = _elu(o + db_ref[0, :][None, :])


def _tail_kernel(c0_ref, c1_ref, cls_ref, ca1_ref, ca2_ref, dw_ref, db_ref,
                 out_ref):
    cls = cls_ref[0, :]
    q = jnp.sum(cls * ca1_ref[0, :])
    e0 = _leaky_relu(q + jnp.sum(cls * ca2_ref[0, :]))
    e1 = _leaky_relu(q + jnp.sum(c0_ref[:, :] * ca2_ref[0, :][None, :], axis=1))
    e2 = _leaky_relu(q + jnp.sum(c1_ref[:, :] * ca2_ref[0, :][None, :], axis=1))
    m = jnp.maximum(jnp.maximum(e0, e1), e2)
    p0 = jnp.exp(e0 - m)
    p1 = jnp.exp(e1 - m)
    p2 = jnp.exp(e2 - m)
    s = p0 + p1 + p2
    pooled = (p0[:, None] * cls[None, :] + p1[:, None] * c0_ref[:, :]
              + p2[:, None] * c1_ref[:, :]) / s[:, None]
    out = jnp.dot(pooled, dw_ref[:, :], preferred_element_type=jnp.float32)
    out_ref[:, :] = _elu(out + db_ref[0, :][None, :])


def kernel(x, adjs, embedding_weight, W0, a0, cls_weight, cls_a, dense_W,
           dense_b):
    k, b, n = x.shape
    heads, embed, hidden = W0.shape
    d = heads * hidden
    pen = dense_W.shape[1]

    x_r = x.reshape(k * b, 1, n)
    adj_r = adjs.reshape(k * b, n, n)
    w_cat = jnp.transpose(W0, (1, 0, 2)).reshape(embed, d)
    a1s = a0[:, :hidden, 0]
    a2s = a0[:, hidden:, 0]
    v1 = jnp.einsum('hef,hf->eh', W0, a1s)
    v2 = jnp.einsum('hef,hf->he', W0, a2s)
    g_bf = (embedding_weight @ w_cat).astype(jnp.bfloat16)
    ev1 = embedding_weight @ v1
    ev2t = (embedding_weight @ v2.T).T

    pairs = 8
    _, out = pl.pallas_call(
        functools.partial(_gat_block_kernel, heads=heads, hidden=hidden,
                          pairs=pairs, nb=b),
        grid=(k * b // pairs,),
        in_specs=[
            pl.BlockSpec((pairs, 1, n), lambda i: (i, 0, 0)),
            pl.BlockSpec((pairs, n, n), lambda i: (i, 0, 0)),
            pl.BlockSpec((n, d), lambda i: (0, 0)),
            pl.BlockSpec((n, heads), lambda i: (0, 0)),
            pl.BlockSpec((heads, n), lambda i: (0, 0)),
            pl.BlockSpec((1, d), lambda i: (0, 0)),
            pl.BlockSpec((1, d), lambda i: (0, 0)),
            pl.BlockSpec((1, d), lambda i: (0, 0)),
            pl.BlockSpec((d, pen), lambda i: (0, 0)),
            pl.BlockSpec((1, pen), lambda i: (0, 0)),
        ],
        out_specs=[
            pl.BlockSpec((k * b, d), lambda i: (0, 0)),
            pl.BlockSpec((b, pen), lambda i: (0, 0)),
        ],
        out_shape=[
            jax.ShapeDtypeStruct((k * b, d), jnp.float32),
            jax.ShapeDtypeStruct((b, pen), jnp.float32),
        ],
        compiler_params=pltpu.CompilerParams(
            dimension_semantics=("arbitrary",)),
    )(x_r, adj_r, g_bf, ev1, ev2t, cls_weight.reshape(1, d),
      cls_a[:d, 0].reshape(1, d), cls_a[d:, 0].reshape(1, d),
      dense_W, dense_b.reshape(1, pen))
    return out


# speedup vs baseline: 1.0595x; 1.0002x over previous
import functools

import jax
import jax.numpy as jnp
from jax.experimental import pallas as pl
from jax.experimental.pallas import tpu as pltpu

ALPHA = 0.2


def _leaky_relu(v):
    return jnp.where(v >= 0, v, ALPHA * v)


def _elu(v):
    return jnp.where(v > 0, v, jnp.exp(v) - 1.0)


def _gat_block_kernel(x_ref, adj_ref, g_ref, ev1_ref, ev2t_ref, cls_ref,
                      ca1_ref, ca2_ref, dw_ref, db_ref, codes_ref, out_ref,
                      *, heads, hidden, pairs, nb):
    n = adj_ref.shape[1]
    i = pl.program_id(0)
    base = i * pairs
    ones_col = jnp.ones((n, 1), dtype=jnp.bfloat16)
    for j in range(pairs):
        xv = x_ref[j, 0, :]
        xvb = xv.astype(jnp.bfloat16)
        adj = adj_ref[j, :, :]
        maskb = jnp.where(adj > 0, 1.0, 0.0).astype(jnp.bfloat16)
        e1c = xv[:, None] * ev1_ref[:, :]
        e2r = ev2t_ref[:, :] * xv[None, :]
        rcol = jnp.exp((ALPHA - 1.0) * e1c).astype(jnp.bfloat16)
        exp_e2 = jnp.exp(e2r).astype(jnp.bfloat16)
        exp_a2 = jnp.exp(ALPHA * e2r).astype(jnp.bfloat16)
        gw = xvb[:, None] * g_ref[:, :]
        code = None
        for h in range(heads):
            ua = rcol[:, h:h + 1] * exp_a2[h:h + 1, :]
            p = jnp.maximum(exp_e2[h:h + 1, :], ua) * maskb
            wh1 = jnp.concatenate(
                [gw[:, h * hidden:(h + 1) * hidden], ones_col], axis=1)
            hs = jnp.dot(p, wh1, preferred_element_type=jnp.float32)
            inv = (1.0 / hs[:, hidden:hidden + 1]).astype(jnp.bfloat16)
            hpb = hs[:, :hidden].astype(jnp.bfloat16) * inv
            hp = jnp.where(hpb > 0, hpb, jnp.exp(hpb) - 1.0)
            gated = xvb[:, None] * hp
            cpart = jnp.max(gated, axis=0)
            code = cpart if code is None else jnp.concatenate([code, cpart])
        codes_ref[base + j, :] = code.astype(jnp.float32)

    @pl.when(i == pl.num_programs(0) - 1)
    def _tail():
        cls = cls_ref[0, :]
        c0 = codes_ref[0:nb, :]
        c1 = codes_ref[nb:2 * nb, :]
        q = jnp.sum(cls * ca1_ref[0, :])
        e0 = _leaky_relu(q + jnp.sum(cls * ca2_ref[0, :]))
        e1 = _leaky_relu(q + jnp.sum(c0 * ca2_ref[0, :][None, :], axis=1))
        e2 = _leaky_relu(q + jnp.sum(c1 * ca2_ref[0, :][None, :], axis=1))
        m = jnp.maximum(jnp.maximum(e0, e1), e2)
        p0 = jnp.exp(e0 - m)
        p1 = jnp.exp(e1 - m)
        p2 = jnp.exp(e2 - m)
        s = p0 + p1 + p2
        pooled = (p0[:, None] * cls[None, :] + p1[:, None] * c0
                  + p2[:, None] * c1) / s[:, None]
        o = jnp.dot(pooled, dw_ref[:, :], preferred_element_type=jnp.float32)
        out_ref[:, :] = _elu(o + db_ref[0, :][None, :])


def kernel(x, adjs, embedding_weight, W0, a0, cls_weight, cls_a, dense_W,
           dense_b):
    k, b, n = x.shape
    heads, embed, hidden = W0.shape
    d = heads * hidden
    pen = dense_W.shape[1]

    x_r = x.reshape(k * b, 1, n)
    adj_r = adjs.reshape(k * b, n, n)
    w_cat = jnp.transpose(W0, (1, 0, 2)).reshape(embed, d)
    a1s = a0[:, :hidden, 0]
    a2s = a0[:, hidden:, 0]
    v1 = jnp.einsum('hef,hf->eh', W0, a1s)
    v2 = jnp.einsum('hef,hf->he', W0, a2s)
    g_bf = (embedding_weight @ w_cat).astype(jnp.bfloat16)
    ev1 = embedding_weight @ v1
    ev2t = (embedding_weight @ v2.T).T

    pairs = 8
    _, out = pl.pallas_call(
        functools.partial(_gat_block_kernel, heads=heads, hidden=hidden,
                          pairs=pairs, nb=b),
        grid=(k * b // pairs,),
        in_specs=[
            pl.BlockSpec((pairs, 1, n), lambda i: (i, 0, 0)),
            pl.BlockSpec((pairs, n, n), lambda i: (i, 0, 0)),
            pl.BlockSpec((n, d), lambda i: (0, 0)),
            pl.BlockSpec((n, heads), lambda i: (0, 0)),
            pl.BlockSpec((heads, n), lambda i: (0, 0)),
            pl.BlockSpec((1, d), lambda i: (0, 0)),
            pl.BlockSpec((1, d), lambda i: (0, 0)),
            pl.BlockSpec((1, d), lambda i: (0, 0)),
            pl.BlockSpec((d, pen), lambda i: (0, 0)),
            pl.BlockSpec((1, pen), lambda i: (0, 0)),
        ],
        out_specs=[
            pl.BlockSpec((k * b, d), lambda i: (0, 0)),
            pl.BlockSpec((b, pen), lambda i: (0, 0)),
        ],
        out_shape=[
            jax.ShapeDtypeStruct((k * b, d), jnp.float32),
            jax.ShapeDtypeStruct((b, pen), jnp.float32),
        ],
        compiler_params=pltpu.CompilerParams(
            dimension_semantics=("arbitrary",)),
    )(x_r, adj_r, g_bf, ev1, ev2t, cls_weight.reshape(1, d),
      cls_a[:d, 0].reshape(1, d), cls_a[d:, 0].reshape(1, d),
      dense_W, dense_b.reshape(1, pen))
    return out
